# baseline (speedup 1.0000x reference)
import functools

import jax
import jax.numpy as jnp
from jax import lax
from jax.experimental import pallas as pl
from jax.experimental.pallas import tpu as pltpu
from jax.experimental.pallas import tpu_sc as plsc

N = 10000
D = 128
E = 320000
NC = 2
NS = 16
NW = NC * NS
C = 128
CH = 80
EPW = C * CH
EPAD = NW * EPW
NPAD = 10240
ZR = NPAD // NS
TCR = 1024
TCG = NPAD // TCR

_mesh = plsc.VectorSubcoreMesh(core_axis_name="c", subcore_axis_name="s")


@functools.partial(
    pl.kernel,
    out_type=jax.ShapeDtypeStruct((NC, NPAD), jnp.float32),
    mesh=_mesh,
    scratch_types=[
        pltpu.VMEM((CH, C), jnp.int32),
        pltpu.VMEM((C,), jnp.float32),
        pltpu.VMEM_SHARED((NPAD,), jnp.float32),
    ],
)
def _deg_kernel(dst_hbm, ones_hbm, zeros_hbm, out_hbm, dst_v, ones_v, acc):
    c = lax.axis_index("c")
    s = lax.axis_index("s")
    wid = c * NS + s
    pltpu.sync_copy(dst_hbm.at[wid], dst_v)
    pltpu.sync_copy(ones_hbm, ones_v)
    pltpu.sync_copy(zeros_hbm, acc.at[pl.ds(s * ZR, ZR)])
    plsc.subcore_barrier()

    def body(j, carry):
        pltpu.sync_copy(ones_v, acc.at[dst_v.at[j]], add=True)
        return carry

    lax.fori_loop(0, CH, body, 0)
    plsc.subcore_barrier()

    @pl.when(s == 0)
    def _():
        pltpu.sync_copy(acc, out_hbm.at[c])


@functools.partial(
    pl.kernel,
    out_type=jax.ShapeDtypeStruct((NC, NPAD, D), jnp.float32),
    mesh=_mesh,
    scratch_types=[
        pltpu.VMEM((CH, C), jnp.int32),
        pltpu.VMEM((CH, C), jnp.int32),
        pltpu.VMEM((C, D), jnp.float32),
        pltpu.VMEM_SHARED((NPAD, D), jnp.float32),
        pltpu.SemaphoreType.DMA,
    ],
)
def _rows_kernel(y_hbm, src_hbm, dst_hbm, zrows_hbm, out_hbm,
                 src_v, dst_v, rows_v, acc, sem):
    c = lax.axis_index("c")
    s = lax.axis_index("s")
    wid = c * NS + s
    pltpu.sync_copy(src_hbm.at[wid], src_v)
    pltpu.sync_copy(dst_hbm.at[wid], dst_v)
    pltpu.sync_copy(zrows_hbm, acc.at[pl.ds(s * ZR, ZR)])
    plsc.subcore_barrier()

    def body(j, carry):
        pltpu.async_copy(y_hbm.at[src_v.at[j]], rows_v, sem).wait()
        pltpu.sync_copy(rows_v, acc.at[dst_v.at[j]], add=True)
        return carry

    lax.fori_loop(0, CH, body, 0)
    plsc.subcore_barrier()
    pltpu.sync_copy(acc.at[pl.ds(s * ZR, ZR)], out_hbm.at[c, pl.ds(s * ZR, ZR)])


def _dinv_block(degp):
    return lax.rsqrt(degp[0] + degp[1] + 1.0)


def _tc1_body(x_ref, wt_ref, degp_ref, y_ref):
    dinv = _dinv_block(degp_ref[...])
    y_ref[...] = jnp.dot(x_ref[...], wt_ref[...],
                         preferred_element_type=jnp.float32) * dinv


def _tc2_body(s_ref, y1_ref, wt_ref, degp_ref, b_ref, y2_ref):
    dinv = _dinv_block(degp_ref[...])
    sp = s_ref[...]
    h = (sp[0] + sp[1] + y1_ref[...]) * dinv + b_ref[...]
    y2_ref[...] = jnp.dot(h, wt_ref[...],
                          preferred_element_type=jnp.float32) * dinv


def _tc3_body(s_ref, y2_ref, degp_ref, b_ref, out_ref):
    dinv = _dinv_block(degp_ref[...])
    sp = s_ref[...]
    out_ref[...] = (sp[0] + sp[1] + y2_ref[...]) * dinv + b_ref[...]


_spec_rows = pl.BlockSpec((TCR, D), lambda i: (i, 0))
_spec_w = pl.BlockSpec((D, D), lambda i: (0, 0))
_spec_deg = pl.BlockSpec((NC, TCR, 1), lambda i: (0, i, 0))
_spec_part = pl.BlockSpec((NC, TCR, D), lambda i: (0, i, 0))
_spec_b = pl.BlockSpec((1, D), lambda i: (0, 0))
_out_rows = jax.ShapeDtypeStruct((NPAD, D), jnp.float32)


def kernel(in_feat, g, W, b):
    src = g[0].astype(jnp.int32)
    dst = g[1].astype(jnp.int32)
    pad = EPAD - E
    src_p = jnp.concatenate([src, jnp.zeros((pad,), jnp.int32)]).reshape(NW, CH, C)
    dst_p = jnp.concatenate([dst, jnp.full((pad,), N, jnp.int32)]).reshape(NW, CH, C)
    x_p = jnp.pad(in_feat, ((0, NPAD - N), (0, 0)))
    Wt = W.T
    b2 = b.reshape(1, D)
    ones_c = jnp.ones((C,), jnp.float32)
    zeros_z = jnp.zeros((ZR,), jnp.float32)
    zrows = jnp.zeros((ZR, D), jnp.float32)

    degp = _deg_kernel(dst_p, ones_c, zeros_z)
    degp3 = degp.reshape(NC, NPAD, 1)

    y1 = pl.pallas_call(
        _tc1_body,
        grid=(TCG,),
        in_specs=[_spec_rows, _spec_w, _spec_deg],
        out_specs=_spec_rows,
        out_shape=_out_rows,
    )(x_p, Wt, degp3)

    s1 = _rows_kernel(y1, src_p, dst_p, zrows)

    y2 = pl.pallas_call(
        _tc2_body,
        grid=(TCG,),
        in_specs=[_spec_part, _spec_rows, _spec_w, _spec_deg, _spec_b],
        out_specs=_spec_rows,
        out_shape=_out_rows,
    )(s1, y1, Wt, degp3, b2)

    s2 = _rows_kernel(y2, src_p, dst_p, zrows)

    out = pl.pallas_call(
        _tc3_body,
        grid=(TCG,),
        in_specs=[_spec_part, _spec_rows, _spec_deg, _spec_b],
        out_specs=_spec_rows,
        out_shape=_out_rows,
    )(s2, y2, degp3, b2)

    return out[:N]


# speedup vs baseline: 1.0017x; 1.0017x over previous
import functools

import jax
import jax.numpy as jnp
from jax import lax
from jax.experimental import pallas as pl
from jax.experimental.pallas import tpu as pltpu
from jax.experimental.pallas import tpu_sc as plsc

N = 10000
D = 128
E = 320000
NC = 2
NS = 16
NW = NC * NS
C = 128
CH = 80
EPW = C * CH
EPAD = NW * EPW
NPAD = 10240
ZR = NPAD // NS
TCR = 1024
TCG = NPAD // TCR

_mesh = plsc.VectorSubcoreMesh(core_axis_name="c", subcore_axis_name="s")


@functools.partial(
    pl.kernel,
    out_type=jax.ShapeDtypeStruct((NC, NPAD), jnp.float32),
    mesh=_mesh,
    scratch_types=[
        pltpu.VMEM((CH, C), jnp.int32),
        pltpu.VMEM((C,), jnp.float32),
        pltpu.VMEM_SHARED((NPAD,), jnp.float32),
    ],
)
def _deg_kernel(dst_hbm, ones_hbm, zeros_hbm, out_hbm, dst_v, ones_v, acc):
    c = lax.axis_index("c")
    s = lax.axis_index("s")
    wid = c * NS + s
    pltpu.sync_copy(dst_hbm.at[wid], dst_v)
    pltpu.sync_copy(ones_hbm, ones_v)
    pltpu.sync_copy(zeros_hbm, acc.at[pl.ds(s * ZR, ZR)])
    plsc.subcore_barrier()

    def body(j, carry):
        pltpu.sync_copy(ones_v, acc.at[dst_v.at[j]], add=True)
        return carry

    lax.fori_loop(0, CH, body, 0)
    plsc.subcore_barrier()

    @pl.when(s == 0)
    def _():
        pltpu.sync_copy(acc, out_hbm.at[c])


@functools.partial(
    pl.kernel,
    out_type=jax.ShapeDtypeStruct((NC, NPAD, D), jnp.float32),
    mesh=_mesh,
    scratch_types=[
        pltpu.VMEM((CH, C), jnp.int32),
        pltpu.VMEM((CH, C), jnp.int32),
        pltpu.VMEM((C, D), jnp.float32),
        pltpu.VMEM_SHARED((NPAD, D), jnp.float32),
        pltpu.SemaphoreType.DMA,
    ],
)
def _rows_kernel(y_hbm, src_hbm, dst_hbm, zrows_hbm, out_hbm,
                 src_v, dst_v, rows_v, acc, sem):
    c = lax.axis_index("c")
    s = lax.axis_index("s")
    wid = c * NS + s
    pltpu.sync_copy(src_hbm.at[wid], src_v)
    pltpu.sync_copy(dst_hbm.at[wid], dst_v)
    pltpu.sync_copy(zrows_hbm, acc.at[pl.ds(s * ZR, ZR)])
    plsc.subcore_barrier()

    def body(j, carry):
        pltpu.async_copy(y_hbm.at[src_v.at[j]], rows_v, sem).wait()
        pltpu.sync_copy(rows_v, acc.at[dst_v.at[j]], add=True)
        return carry

    lax.fori_loop(0, CH, body, 0)
    plsc.subcore_barrier()
    pltpu.sync_copy(acc.at[pl.ds(s * ZR, ZR)], out_hbm.at[c, pl.ds(s * ZR, ZR)])


def _dinv_block(degp):
    return lax.rsqrt(degp[0] + degp[1] + 1.0)


def _tc1_body(x_ref, wt_ref, degp_ref, y_ref):
    dinv = _dinv_block(degp_ref[...])
    y_ref[...] = jnp.dot(x_ref[...], wt_ref[...],
                         preferred_element_type=jnp.float32) * dinv


def _tc2_body(s_ref, y1_ref, wt_ref, degp_ref, b_ref, y2_ref):
    dinv = _dinv_block(degp_ref[...])
    sp = s_ref[...]
    h = (sp[0] + sp[1] + y1_ref[...]) * dinv + b_ref[...]
    y2_ref[...] = jnp.dot(h, wt_ref[...],
                          preferred_element_type=jnp.float32) * dinv


def _tc3_body(s_ref, y2_ref, degp_ref, b_ref, out_ref):
    dinv = _dinv_block(degp_ref[...])
    sp = s_ref[...]
    out_ref[...] = (sp[0] + sp[1] + y2_ref[...]) * dinv + b_ref[...]


_spec_rows = pl.BlockSpec((TCR, D), lambda i: (i, 0))
_spec_w = pl.BlockSpec((D, D), lambda i: (0, 0))
_spec_deg = pl.BlockSpec((NC, TCR, 1), lambda i: (0, i, 0))
_spec_part = pl.BlockSpec((NC, TCR, D), lambda i: (0, i, 0))
_spec_b = pl.BlockSpec((1, D), lambda i: (0, 0))
_out_rows = jax.ShapeDtypeStruct((NPAD, D), jnp.float32)


def kernel(in_feat, g, W, b):
    src = g[0].astype(jnp.int32)
    dst = g[1].astype(jnp.int32)
    pad = EPAD - E
    src_p = jnp.concatenate([src, jnp.zeros((pad,), jnp.int32)]).reshape(NW, CH, C)
    trash = N + (jnp.arange(pad, dtype=jnp.int32) % (NPAD - N))
    dst_p = jnp.concatenate([dst, trash]).reshape(NW, CH, C)
    x_p = jnp.pad(in_feat, ((0, NPAD - N), (0, 0)))
    Wt = W.T
    b2 = b.reshape(1, D)
    ones_c = jnp.ones((C,), jnp.float32)
    zeros_z = jnp.zeros((ZR,), jnp.float32)
    zrows = jnp.zeros((ZR, D), jnp.float32)

    degp = _deg_kernel(dst_p, ones_c, zeros_z)
    degp3 = degp.reshape(NC, NPAD, 1)

    y1 = pl.pallas_call(
        _tc1_body,
        grid=(TCG,),
        in_specs=[_spec_rows, _spec_w, _spec_deg],
        out_specs=_spec_rows,
        out_shape=_out_rows,
    )(x_p, Wt, degp3)

    s1 = _rows_kernel(y1, src_p, dst_p, zrows)

    y2 = pl.pallas_call(
        _tc2_body,
        grid=(TCG,),
        in_specs=[_spec_part, _spec_rows, _spec_w, _spec_deg, _spec_b],
        out_specs=_spec_rows,
        out_shape=_out_rows,
    )(s1, y1, Wt, degp3, b2)

    s2 = _rows_kernel(y2, src_p, dst_p, zrows)

    out = pl.pallas_call(
        _tc3_body,
        grid=(TCG,),
        in_specs=[_spec_part, _spec_rows, _spec_deg, _spec_b],
        out_specs=_spec_rows,
        out_shape=_out_rows,
    )(s2, y2, degp3, b2)

    return out[:N]


# speedup vs baseline: 2.5023x; 2.4980x over previous
import functools

import jax
import jax.numpy as jnp
from jax import lax
from jax.experimental import pallas as pl
from jax.experimental.pallas import tpu as pltpu
from jax.experimental.pallas import tpu_sc as plsc

N = 10000
D = 128
E = 320000
NC = 2
NS = 16
NW = NC * NS
C = 128
CH = 80
EPW = C * CH
EPAD = NW * EPW
NPAD = 10240
ZR = NPAD // NS
TCR = 1024
TCG = NPAD // TCR

_mesh = plsc.VectorSubcoreMesh(core_axis_name="c", subcore_axis_name="s")


@functools.partial(
    pl.kernel,
    out_type=jax.ShapeDtypeStruct((NC, NPAD), jnp.float32),
    mesh=_mesh,
    scratch_types=[
        pltpu.VMEM((CH, C), jnp.int32),
        pltpu.VMEM((C,), jnp.float32),
        pltpu.VMEM_SHARED((NPAD,), jnp.float32),
    ],
)
def _deg_kernel(dst_hbm, ones_hbm, zeros_hbm, out_hbm, dst_v, ones_v, acc):
    c = lax.axis_index("c")
    s = lax.axis_index("s")
    wid = c * NS + s
    pltpu.sync_copy(dst_hbm.at[wid], dst_v)
    pltpu.sync_copy(ones_hbm, ones_v)
    pltpu.sync_copy(zeros_hbm, acc.at[pl.ds(s * ZR, ZR)])
    plsc.subcore_barrier()

    def body(j, carry):
        pltpu.sync_copy(ones_v, acc.at[dst_v.at[j]], add=True)
        return carry

    lax.fori_loop(0, CH, body, 0)
    plsc.subcore_barrier()

    @pl.when(s == 0)
    def _():
        pltpu.sync_copy(acc, out_hbm.at[c])


@functools.partial(
    pl.kernel,
    out_type=jax.ShapeDtypeStruct((NC, NPAD, D), jnp.float32),
    mesh=_mesh,
    scratch_types=[
        pltpu.VMEM((CH, C), jnp.int32),
        pltpu.VMEM((CH, C), jnp.int32),
        pltpu.VMEM((C, D), jnp.float32),
        pltpu.VMEM_SHARED((NPAD, D), jnp.float32),
        pltpu.SemaphoreType.DMA,
    ],
)
def _rows_kernel(y_hbm, src_hbm, dst_hbm, zrows_hbm, out_hbm,
                 src_v, dst_v, rows_v, acc, sem):
    c = lax.axis_index("c")
    s = lax.axis_index("s")
    wid = c * NS + s
    pltpu.sync_copy(src_hbm.at[wid], src_v)
    pltpu.sync_copy(dst_hbm.at[wid], dst_v)
    pltpu.sync_copy(zrows_hbm, acc.at[pl.ds(s * ZR, ZR)])
    plsc.subcore_barrier()

    def body(j, carry):
        pltpu.async_copy(y_hbm.at[src_v.at[j]], rows_v, sem).wait()
        pltpu.sync_copy(rows_v, acc.at[dst_v.at[j]], add=True)
        return carry

    lax.fori_loop(0, CH, body, 0)
    plsc.subcore_barrier()
    pltpu.sync_copy(acc.at[pl.ds(s * ZR, ZR)], out_hbm.at[c, pl.ds(s * ZR, ZR)])


def _dinv_block(degp):
    return lax.rsqrt(degp[0] + degp[1] + 1.0)


def _tc1_body(x_ref, wt_ref, degp_ref, y_ref):
    dinv = _dinv_block(degp_ref[...])
    y_ref[...] = jnp.dot(x_ref[...], wt_ref[...],
                         preferred_element_type=jnp.float32) * dinv


def _tc2_body(s_ref, y1_ref, wt_ref, degp_ref, b_ref, y2_ref):
    dinv = _dinv_block(degp_ref[...])
    sp = s_ref[...]
    h = (sp[0] + sp[1] + y1_ref[...]) * dinv + b_ref[...]
    y2_ref[...] = jnp.dot(h, wt_ref[...],
                          preferred_element_type=jnp.float32) * dinv


def _tc3_body(s_ref, y2_ref, degp_ref, b_ref, out_ref):
    dinv = _dinv_block(degp_ref[...])
    sp = s_ref[...]
    out_ref[...] = (sp[0] + sp[1] + y2_ref[...]) * dinv + b_ref[...]


_spec_rows = pl.BlockSpec((TCR, D), lambda i: (i, 0))
_spec_w = pl.BlockSpec((D, D), lambda i: (0, 0))
_spec_deg = pl.BlockSpec((NC, TCR, 1), lambda i: (0, i, 0))
_spec_part = pl.BlockSpec((NC, TCR, D), lambda i: (0, i, 0))
_spec_b = pl.BlockSpec((1, D), lambda i: (0, 0))
_out_rows = jax.ShapeDtypeStruct((NPAD, D), jnp.float32)


def kernel(in_feat, g, W, b):
    src = g[0].astype(jnp.int32)
    dst = g[1].astype(jnp.int32)
    pad = EPAD - E
    psrc = jnp.arange(pad, dtype=jnp.int32) % N
    src_p = jnp.concatenate([src, psrc]).reshape(NW, CH, C)
    trash = N + (jnp.arange(pad, dtype=jnp.int32) % (NPAD - N))
    dst_p = jnp.concatenate([dst, trash]).reshape(NW, CH, C)
    x_p = jnp.pad(in_feat, ((0, NPAD - N), (0, 0)))
    Wt = W.T
    b2 = b.reshape(1, D)
    ones_c = jnp.ones((C,), jnp.float32)
    zeros_z = jnp.zeros((ZR,), jnp.float32)
    zrows = jnp.zeros((ZR, D), jnp.float32)

    degp = _deg_kernel(dst_p, ones_c, zeros_z)
    degp3 = degp.reshape(NC, NPAD, 1)

    y1 = pl.pallas_call(
        _tc1_body,
        grid=(TCG,),
        in_specs=[_spec_rows, _spec_w, _spec_deg],
        out_specs=_spec_rows,
        out_shape=_out_rows,
    )(x_p, Wt, degp3)

    s1 = _rows_kernel(y1, src_p, dst_p, zrows)

    y2 = pl.pallas_call(
        _tc2_body,
        grid=(TCG,),
        in_specs=[_spec_part, _spec_rows, _spec_w, _spec_deg, _spec_b],
        out_specs=_spec_rows,
        out_shape=_out_rows,
    )(s1, y1, Wt, degp3, b2)

    s2 = _rows_kernel(y2, src_p, dst_p, zrows)

    out = pl.pallas_call(
        _tc3_body,
        grid=(TCG,),
        in_specs=[_spec_part, _spec_rows, _spec_deg, _spec_b],
        out_specs=_spec_rows,
        out_shape=_out_rows,
    )(s2, y2, degp3, b2)

    return out[:N]
